# Initial kernel scaffold; baseline (speedup 1.0000x reference)
import functools

import jax
import jax.numpy as jnp
from jax import lax
from jax.experimental import pallas as pl
from jax.experimental.pallas import tpu as pltpu
from jax.experimental.pallas import tpu_sc as plsc

N = 100000
E = 6400000
SIGMA = 0.1
EPSILON = 1.0
CUTOFF = 2.5

NC = 2
NS = 16
L = 16
NW = NC * NS
ROWW = 16
GB = 128
NGB = 8
CHUNK = NGB * GB
CPW = 196
E_PAD = NW * CPW * CHUNK
SENT_I = N
SENT_J = N + 1
N_PAD = 100096
RPS = N_PAD // NS

SIG2 = SIGMA * SIGMA
CUT2 = CUTOFF * CUTOFF
FCONST = 24.0 * EPSILON / SIG2

_mesh = plsc.VectorSubcoreMesh(core_axis_name="c", subcore_axis_name="s")


@functools.partial(
    pl.kernel,
    out_type=(jax.ShapeDtypeStruct((NC, N_PAD, ROWW), jnp.float32),
              jax.ShapeDtypeStruct((NW, L), jnp.float32)),
    mesh=_mesh,
    scratch_types=[
        pltpu.VMEM((NGB, GB), jnp.int32),
        pltpu.VMEM((NGB, GB), jnp.int32),
        pltpu.VMEM((CHUNK, ROWW), jnp.float32),
        pltpu.VMEM((CHUNK, ROWW), jnp.float32),
        pltpu.VMEM((CHUNK, ROWW), jnp.float32),
        pltpu.VMEM((CHUNK, ROWW), jnp.float32),
        pltpu.VMEM((L,), jnp.float32),
        pltpu.VMEM_SHARED((N_PAD, ROWW), jnp.float32),
        pltpu.SemaphoreType.DMA,
    ],
)
def _lj_edges(pos_hbm, src_hbm, dst_hbm, zero_hbm, fpart_hbm, epart_hbm,
              idx_i, idx_j, rows_i, rows_j, fpos, fneg, eacc, facc, gsem):
    cid = lax.axis_index("c")
    sid = lax.axis_index("s")
    wid = sid * NC + cid

    pltpu.sync_copy(zero_hbm.at[pl.ds(sid * RPS, RPS)],
                    facc.at[pl.ds(sid * RPS, RPS)])
    pltpu.sync_copy(zero_hbm.at[pl.ds(0, CHUNK)], fpos)
    pltpu.sync_copy(zero_hbm.at[pl.ds(0, CHUNK)], fneg)
    eacc[...] = jnp.zeros((L,), jnp.float32)
    plsc.subcore_barrier()

    lane = lax.iota(jnp.int32, L)
    c0 = jnp.zeros((L,), jnp.int32)
    c1 = jnp.full((L,), 1, jnp.int32)
    c2 = jnp.full((L,), 2, jnp.int32)

    row0 = wid * (CPW * NGB)

    @pl.loop(0, CPW)
    def _chunk(t):
        r0 = row0 + t * NGB
        pltpu.sync_copy(src_hbm.at[pl.ds(r0, NGB)], idx_i)
        pltpu.sync_copy(dst_hbm.at[pl.ds(r0, NGB)], idx_j)
        gets = []
        for g in range(NGB):
            gets.append(pltpu.async_copy(pos_hbm.at[idx_i.at[g]],
                                         rows_i.at[pl.ds(g * GB, GB)], gsem))
            gets.append(pltpu.async_copy(pos_hbm.at[idx_j.at[g]],
                                         rows_j.at[pl.ds(g * GB, GB)], gsem))
        for h in gets:
            h.wait()

        @pl.loop(0, CHUNK, step=L)
        def _grp(e0):
            ridx = lane + e0
            xi = plsc.load_gather(rows_i, [ridx, c0])
            yi = plsc.load_gather(rows_i, [ridx, c1])
            zi = plsc.load_gather(rows_i, [ridx, c2])
            xj = plsc.load_gather(rows_j, [ridx, c0])
            yj = plsc.load_gather(rows_j, [ridx, c1])
            zj = plsc.load_gather(rows_j, [ridx, c2])
            dx = xj - xi
            dy = yj - yi
            dz = zj - zi
            r2 = dx * dx + dy * dy + dz * dz
            s2 = SIG2 / r2
            s6 = s2 * s2 * s2
            s12 = s6 * s6
            mask = r2 < CUT2
            zero = jnp.zeros((L,), jnp.float32)
            e = jnp.where(mask, (4.0 * EPSILON) * (s12 - s6), zero)
            plsc.addupdate(eacc, e)
            cf = jnp.where(mask, (FCONST * s2) * (s12 + s12 - s6), zero)
            fx = cf * dx
            fy = cf * dy
            fz = cf * dz
            plsc.store_scatter(fpos, [ridx, c0], fx)
            plsc.store_scatter(fpos, [ridx, c1], fy)
            plsc.store_scatter(fpos, [ridx, c2], fz)
            plsc.store_scatter(fneg, [ridx, c0], -fx)
            plsc.store_scatter(fneg, [ridx, c1], -fy)
            plsc.store_scatter(fneg, [ridx, c2], -fz)

        for g in range(NGB):
            pltpu.sync_copy(fpos.at[pl.ds(g * GB, GB)],
                            facc.at[idx_j.at[g]], add=True)
            pltpu.sync_copy(fneg.at[pl.ds(g * GB, GB)],
                            facc.at[idx_i.at[g]], add=True)

    plsc.subcore_barrier()
    pltpu.sync_copy(facc.at[pl.ds(sid * RPS, RPS)],
                    fpart_hbm.at[cid, pl.ds(sid * RPS, RPS)])
    pltpu.sync_copy(eacc, epart_hbm.at[wid])


MR = N_PAD * ROWW // 128


def _merge_body(f_ref, e_ref, o_ref, es_ref):
    o_ref[...] = f_ref[0] + f_ref[1]
    es_ref[...] = jnp.broadcast_to(0.5 * jnp.sum(e_ref[...]), (1, 1))


def kernel(positions, edge_index):
    src = edge_index[0]
    dst = edge_index[1]
    pad = E_PAD - E
    srcp = jnp.concatenate(
        [src, jnp.full((pad,), SENT_I, jnp.int32)]).reshape(-1, GB)
    dstp = jnp.concatenate(
        [dst, jnp.full((pad,), SENT_J, jnp.int32)]).reshape(-1, GB)
    pos16 = jnp.zeros((N_PAD, ROWW), jnp.float32).at[:N, :3].set(positions)
    pos16 = pos16.at[SENT_J, 0].set(1e6)
    z16 = jnp.zeros((N_PAD, ROWW), jnp.float32)
    fpart, epart = _lj_edges(pos16, srcp, dstp, z16)
    out128, esum = pl.pallas_call(
        _merge_body,
        out_shape=(jax.ShapeDtypeStruct((MR, 128), jnp.float32),
                   jax.ShapeDtypeStruct((1, 1), jnp.float32)),
    )(fpart.reshape(NC, MR, 128), epart)
    forces = out128.reshape(N_PAD, ROWW)[:N, :3]
    energy = esum[0, 0]
    return energy, forces


# speedup vs baseline: 37.1593x; 37.1593x over previous
import dataclasses
import functools

import jax
import jax.numpy as jnp
from jax import lax
from jax.experimental import pallas as pl
from jax.experimental.pallas import tpu as pltpu
from jax.experimental.pallas import tpu_sc as plsc

N = 100000
E = 6400000
SIGMA = 0.1
EPSILON = 1.0
CUTOFF = 2.5

NC = 2
NS = 16
L = 16
NW = NC * NS
ROWW = 16
FW = 8
GB = 128
NGB = 8
CHUNK = NGB * GB
CPW = 196
E_PAD = NW * CPW * CHUNK
SENT_I = N
SENT_J = N + 1
N_PAD = 100096
RPS = N_PAD // NS

SIG2 = SIGMA * SIGMA
CUT2 = CUTOFF * CUTOFF
FCONST = 24.0 * EPSILON / SIG2

_mesh = plsc.VectorSubcoreMesh(core_axis_name="c", subcore_axis_name="s")

_cp = pltpu.CompilerParams()
if "needs_layout_passes" in pltpu.CompilerParams.__dataclass_fields__:
    _cp = dataclasses.replace(_cp, needs_layout_passes=False)
if "use_tc_tiling_on_sc" in pltpu.CompilerParams.__dataclass_fields__:
    _cp = dataclasses.replace(_cp, use_tc_tiling_on_sc=False)


@functools.partial(
    pl.kernel,
    compiler_params=_cp,
    out_type=(jax.ShapeDtypeStruct((NC, N_PAD, FW), jnp.float32),
              jax.ShapeDtypeStruct((NW, L), jnp.float32)),
    mesh=_mesh,
    scratch_types=[
        pltpu.VMEM((NGB, GB), jnp.int32),
        pltpu.VMEM((NGB, GB), jnp.int32),
        pltpu.VMEM((CHUNK, ROWW), jnp.float32),
        pltpu.VMEM((CHUNK, ROWW), jnp.float32),
        pltpu.VMEM((CHUNK, FW), jnp.float32),
        pltpu.VMEM((CHUNK, FW), jnp.float32),
        pltpu.VMEM((L,), jnp.float32),
        pltpu.VMEM_SHARED((N_PAD, FW), jnp.float32),
        pltpu.SemaphoreType.DMA,
    ],
)
def _lj_edges(pos_hbm, src_hbm, dst_hbm, zero_hbm, fpart_hbm, epart_hbm,
              idx_i, idx_j, rows_i, rows_j, fpos, fneg, eacc, facc, gsem):
    cid = lax.axis_index("c")
    sid = lax.axis_index("s")
    wid = sid * NC + cid

    pltpu.sync_copy(zero_hbm.at[pl.ds(sid * RPS, RPS)],
                    facc.at[pl.ds(sid * RPS, RPS)])
    pltpu.sync_copy(zero_hbm.at[pl.ds(0, CHUNK)], fpos)
    pltpu.sync_copy(zero_hbm.at[pl.ds(0, CHUNK)], fneg)
    eacc[...] = jnp.zeros((L,), jnp.float32)
    plsc.subcore_barrier()

    lane = lax.iota(jnp.int32, L)
    c0 = jnp.zeros((L,), jnp.int32)
    c1 = jnp.full((L,), 1, jnp.int32)
    c2 = jnp.full((L,), 2, jnp.int32)

    row0 = wid * (CPW * NGB)

    @pl.loop(0, CPW)
    def _chunk(t):
        r0 = row0 + t * NGB
        pltpu.sync_copy(src_hbm.at[pl.ds(r0, NGB)], idx_i)
        pltpu.sync_copy(dst_hbm.at[pl.ds(r0, NGB)], idx_j)
        gets = []
        for g in range(NGB):
            gets.append(pltpu.async_copy(pos_hbm.at[idx_i.at[g]],
                                         rows_i.at[pl.ds(g * GB, GB)], gsem))
            gets.append(pltpu.async_copy(pos_hbm.at[idx_j.at[g]],
                                         rows_j.at[pl.ds(g * GB, GB)], gsem))
        for h in gets:
            h.wait()

        @pl.loop(0, CHUNK, step=L)
        def _grp(e0):
            ridx = lane + e0
            xi = plsc.load_gather(rows_i, [ridx, c0])
            yi = plsc.load_gather(rows_i, [ridx, c1])
            zi = plsc.load_gather(rows_i, [ridx, c2])
            xj = plsc.load_gather(rows_j, [ridx, c0])
            yj = plsc.load_gather(rows_j, [ridx, c1])
            zj = plsc.load_gather(rows_j, [ridx, c2])
            dx = xj - xi
            dy = yj - yi
            dz = zj - zi
            r2 = dx * dx + dy * dy + dz * dz
            s2 = SIG2 / r2
            s6 = s2 * s2 * s2
            s12 = s6 * s6
            mask = r2 < CUT2
            zero = jnp.zeros((L,), jnp.float32)
            e = jnp.where(mask, (4.0 * EPSILON) * (s12 - s6), zero)
            eacc[...] += e
            cf = jnp.where(mask, (FCONST * s2) * (s12 + s12 - s6), zero)
            fx = cf * dx
            fy = cf * dy
            fz = cf * dz
            plsc.store_scatter(fpos, [ridx, c0], fx)
            plsc.store_scatter(fpos, [ridx, c1], fy)
            plsc.store_scatter(fpos, [ridx, c2], fz)
            plsc.store_scatter(fneg, [ridx, c0], -fx)
            plsc.store_scatter(fneg, [ridx, c1], -fy)
            plsc.store_scatter(fneg, [ridx, c2], -fz)

        for g in range(NGB):
            pltpu.sync_copy(fpos.at[pl.ds(g * GB, GB)],
                            facc.at[idx_j.at[g]], add=True)
            pltpu.sync_copy(fneg.at[pl.ds(g * GB, GB)],
                            facc.at[idx_i.at[g]], add=True)

    plsc.subcore_barrier()
    pltpu.sync_copy(facc.at[pl.ds(sid * RPS, RPS)],
                    fpart_hbm.at[cid, pl.ds(sid * RPS, RPS)])
    pltpu.sync_copy(eacc, epart_hbm.at[wid])


MR = N_PAD * FW // 128


def _merge_body(f_ref, e_ref, o_ref, es_ref):
    o_ref[...] = f_ref[0] + f_ref[1]
    es_ref[...] = jnp.broadcast_to(0.5 * jnp.sum(e_ref[...]), (1, 1))


def kernel(positions, edge_index):
    src = edge_index[0]
    dst = edge_index[1]
    pad = E_PAD - E
    srcp = jnp.concatenate(
        [src, jnp.full((pad,), SENT_I, jnp.int32)]).reshape(-1, GB)
    dstp = jnp.concatenate(
        [dst, jnp.full((pad,), SENT_J, jnp.int32)]).reshape(-1, GB)
    pos16 = jnp.zeros((N_PAD, ROWW), jnp.float32).at[:N, :3].set(positions)
    pos16 = pos16.at[SENT_J, 0].set(1e6)
    z16 = jnp.zeros((N_PAD, FW), jnp.float32)
    fpart, epart = _lj_edges(pos16, srcp, dstp, z16)
    out128, esum = pl.pallas_call(
        _merge_body,
        out_shape=(jax.ShapeDtypeStruct((MR, 128), jnp.float32),
                   jax.ShapeDtypeStruct((1, 1), jnp.float32)),
    )(fpart.reshape(NC, MR, 128), epart)
    forces = out128.reshape(N_PAD, FW)[:N, :3]
    energy = esum[0, 0]
    return energy, forces


# speedup vs baseline: 67.1585x; 1.8073x over previous
import dataclasses
import functools

import jax
import jax.numpy as jnp
from jax import lax
from jax.experimental import pallas as pl
from jax.experimental.pallas import tpu as pltpu
from jax.experimental.pallas import tpu_sc as plsc

N = 100000
E = 6400000
SIGMA = 0.1
EPSILON = 1.0
CUTOFF = 2.5

NC = 2
NS = 16
L = 16
NW = NC * NS
ROWW = 16
FW = 8
GB = 128
NGB = 4
CHUNK = NGB * GB
CPW = 392
E_PAD = NW * CPW * CHUNK
SENT_I = N
SENT_J = N + 1
N_PAD = 100096
RPS = N_PAD // NS

SIG2 = SIGMA * SIGMA
CUT2 = CUTOFF * CUTOFF
FCONST = 24.0 * EPSILON / SIG2

_mesh = plsc.VectorSubcoreMesh(core_axis_name="c", subcore_axis_name="s")

_cp = pltpu.CompilerParams()
if "needs_layout_passes" in pltpu.CompilerParams.__dataclass_fields__:
    _cp = dataclasses.replace(_cp, needs_layout_passes=False)
if "use_tc_tiling_on_sc" in pltpu.CompilerParams.__dataclass_fields__:
    _cp = dataclasses.replace(_cp, use_tc_tiling_on_sc=False)

_scratch = []
for _ in range(4):
    _scratch += [pltpu.VMEM((NGB, GB), jnp.int32),
                 pltpu.VMEM((NGB, GB), jnp.int32),
                 pltpu.SemaphoreType.DMA]
for _ in range(2):
    _scratch += [pltpu.VMEM((CHUNK, ROWW), jnp.float32),
                 pltpu.VMEM((CHUNK, ROWW), jnp.float32),
                 pltpu.SemaphoreType.DMA]
for _ in range(2):
    _scratch += [pltpu.VMEM((CHUNK, FW), jnp.float32),
                 pltpu.VMEM((CHUNK, FW), jnp.float32),
                 pltpu.SemaphoreType.DMA]
_scratch += [pltpu.VMEM((L,), jnp.float32),
             pltpu.VMEM_SHARED((N_PAD, FW), jnp.float32)]


@functools.partial(
    pl.kernel,
    compiler_params=_cp,
    out_type=(jax.ShapeDtypeStruct((NC, N_PAD, FW), jnp.float32),
              jax.ShapeDtypeStruct((NW, L), jnp.float32)),
    mesh=_mesh,
    scratch_types=_scratch,
)
def _lj_edges(pos_hbm, src_hbm, dst_hbm, zero_hbm, fpart_hbm, epart_hbm,
              *scr):
    idxs = [scr[3 * k:3 * k + 3] for k in range(4)]
    rowss = [scr[12 + 3 * k:12 + 3 * k + 3] for k in range(2)]
    fs = [scr[18 + 3 * k:18 + 3 * k + 3] for k in range(2)]
    eacc = scr[24]
    facc = scr[25]

    cid = lax.axis_index("c")
    sid = lax.axis_index("s")
    wid = sid * NC + cid

    pltpu.sync_copy(zero_hbm.at[pl.ds(sid * RPS, RPS)],
                    facc.at[pl.ds(sid * RPS, RPS)])
    for fp, fn, _ in fs:
        pltpu.sync_copy(zero_hbm.at[pl.ds(0, CHUNK)], fp)
        pltpu.sync_copy(zero_hbm.at[pl.ds(0, CHUNK)], fn)
    eacc[...] = jnp.zeros((L,), jnp.float32)
    plsc.subcore_barrier()

    lane = lax.iota(jnp.int32, L)
    c0 = jnp.zeros((L,), jnp.int32)
    c1 = jnp.full((L,), 1, jnp.int32)
    c2 = jnp.full((L,), 2, jnp.int32)

    row0 = wid * (CPW * NGB)

    def idx_descs(t, si):
        ii, ij, isem = idxs[si]
        r0 = row0 + t * NGB
        return [pltpu.make_async_copy(src_hbm.at[pl.ds(r0, NGB)], ii, isem),
                pltpu.make_async_copy(dst_hbm.at[pl.ds(r0, NGB)], ij, isem)]

    def gather_descs(si, sr):
        ii, ij, _ = idxs[si]
        ri, rj, gsem = rowss[sr]
        ds = []
        for g in range(NGB):
            ds.append(pltpu.make_async_copy(
                pos_hbm.at[ii.at[g]], ri.at[pl.ds(g * GB, GB)], gsem))
            ds.append(pltpu.make_async_copy(
                pos_hbm.at[ij.at[g]], rj.at[pl.ds(g * GB, GB)], gsem))
        return ds

    def issue_scatter(si, sf):
        ii, ij, _ = idxs[si]
        fp, fn, ssem = fs[sf]
        for g in range(NGB):
            pltpu.async_copy(fp.at[pl.ds(g * GB, GB)],
                             facc.at[ij.at[g]], ssem, add=True)
            pltpu.async_copy(fn.at[pl.ds(g * GB, GB)],
                             facc.at[ii.at[g]], ssem, add=True)

    def wait_scatter(sf):
        fp, fn, ssem = fs[sf]
        pltpu.make_async_copy(zero_hbm.at[pl.ds(0, CHUNK)], fp, ssem).wait()
        pltpu.make_async_copy(zero_hbm.at[pl.ds(0, CHUNK)], fn, ssem).wait()

    def compute(sr, sf):
        ri, rj, _ = rowss[sr]
        fp, fn, _ = fs[sf]

        @pl.loop(0, CHUNK, step=L)
        def _grp(e0):
            ridx = lane + e0
            xi = plsc.load_gather(ri, [ridx, c0])
            yi = plsc.load_gather(ri, [ridx, c1])
            zi = plsc.load_gather(ri, [ridx, c2])
            xj = plsc.load_gather(rj, [ridx, c0])
            yj = plsc.load_gather(rj, [ridx, c1])
            zj = plsc.load_gather(rj, [ridx, c2])
            dx = xj - xi
            dy = yj - yi
            dz = zj - zi
            r2 = dx * dx + dy * dy + dz * dz
            s2 = SIG2 / r2
            s6 = s2 * s2 * s2
            s12 = s6 * s6
            mask = r2 < CUT2
            zero = jnp.zeros((L,), jnp.float32)
            e = jnp.where(mask, (4.0 * EPSILON) * (s12 - s6), zero)
            eacc[...] += e
            cf = jnp.where(mask, (FCONST * s2) * (s12 + s12 - s6), zero)
            fx = cf * dx
            fy = cf * dy
            fz = cf * dz
            plsc.store_scatter(fp, [ridx, c0], fx)
            plsc.store_scatter(fp, [ridx, c1], fy)
            plsc.store_scatter(fp, [ridx, c2], fz)
            plsc.store_scatter(fn, [ridx, c0], -fx)
            plsc.store_scatter(fn, [ridx, c1], -fy)
            plsc.store_scatter(fn, [ridx, c2], -fz)

    def phase(t, si, sr, wg1=True, wsc=True, pi2=True):
        if wg1:
            for d in idx_descs(t + 1, (si + 1) % 4):
                d.wait()
            for d in gather_descs((si + 1) % 4, (sr + 1) % 2):
                d.start()
        if wsc:
            wait_scatter(sr)
        if pi2:
            for d in idx_descs(t + 2, (si + 2) % 4):
                d.start()
        for d in gather_descs(si, sr):
            d.wait()
        compute(sr, sr)
        issue_scatter(si, sr)

    for d in idx_descs(0, 0):
        d.start()
        d.wait()
    for d in gather_descs(0, 0):
        d.start()
    for d in idx_descs(1, 1):
        d.start()

    phase(0, 0, 0, wsc=False)
    phase(1, 1, 1, wsc=False)
    phase(2, 2, 0)
    phase(3, 3, 1)

    @pl.loop(4, CPW - 4, step=4)
    def _main(tb):
        phase(tb + 0, 0, 0)
        phase(tb + 1, 1, 1)
        phase(tb + 2, 2, 0)
        phase(tb + 3, 3, 1)

    phase(CPW - 4, 0, 0)
    phase(CPW - 3, 1, 1)
    phase(CPW - 2, 2, 0, pi2=False)
    phase(CPW - 1, 3, 1, wg1=False, pi2=False)
    wait_scatter(0)
    wait_scatter(1)

    plsc.subcore_barrier()
    pltpu.sync_copy(facc.at[pl.ds(sid * RPS, RPS)],
                    fpart_hbm.at[cid, pl.ds(sid * RPS, RPS)])
    pltpu.sync_copy(eacc, epart_hbm.at[wid])


MR = N_PAD * FW // 128


def _merge_body(f_ref, e_ref, o_ref, es_ref):
    o_ref[...] = f_ref[0] + f_ref[1]
    es_ref[...] = jnp.broadcast_to(0.5 * jnp.sum(e_ref[...]), (1, 1))


def kernel(positions, edge_index):
    src = edge_index[0]
    dst = edge_index[1]
    pad = E_PAD - E
    srcp = jnp.concatenate(
        [src, jnp.full((pad,), SENT_I, jnp.int32)]).reshape(-1, GB)
    dstp = jnp.concatenate(
        [dst, jnp.full((pad,), SENT_J, jnp.int32)]).reshape(-1, GB)
    pos16 = jnp.zeros((N_PAD, ROWW), jnp.float32).at[:N, :3].set(positions)
    pos16 = pos16.at[SENT_J, 0].set(1e6)
    z16 = jnp.zeros((N_PAD, FW), jnp.float32)
    fpart, epart = _lj_edges(pos16, srcp, dstp, z16)
    out128, esum = pl.pallas_call(
        _merge_body,
        out_shape=(jax.ShapeDtypeStruct((MR, 128), jnp.float32),
                   jax.ShapeDtypeStruct((1, 1), jnp.float32)),
    )(fpart.reshape(NC, MR, 128), epart)
    forces = out128.reshape(N_PAD, FW)[:N, :3]
    energy = esum[0, 0]
    return energy, forces


# speedup vs baseline: 68.2132x; 1.0157x over previous
import dataclasses
import functools

import jax
import jax.numpy as jnp
from jax import lax
from jax.experimental import pallas as pl
from jax.experimental.pallas import tpu as pltpu
from jax.experimental.pallas import tpu_sc as plsc

N = 100000
E = 6400000
SIGMA = 0.1
EPSILON = 1.0
CUTOFF = 2.5

NC = 2
NS = 16
L = 16
NW = NC * NS
ROWW = 16
FW = 8
GB = 128
NGB = 4
CHUNK = NGB * GB
CPW = 392
E_PAD = NW * CPW * CHUNK
SENT_I = N
SENT_J = N + 1
N_PAD = 100096
RPS = N_PAD // NS

SIG2 = SIGMA * SIGMA
CUT2 = CUTOFF * CUTOFF
FCONST = 24.0 * EPSILON / SIG2

_mesh = plsc.VectorSubcoreMesh(core_axis_name="c", subcore_axis_name="s")

_cp = pltpu.CompilerParams()
if "needs_layout_passes" in pltpu.CompilerParams.__dataclass_fields__:
    _cp = dataclasses.replace(_cp, needs_layout_passes=False)
if "use_tc_tiling_on_sc" in pltpu.CompilerParams.__dataclass_fields__:
    _cp = dataclasses.replace(_cp, use_tc_tiling_on_sc=False)

_scratch = []
for _ in range(4):
    _scratch += [pltpu.VMEM((NGB, GB), jnp.int32),
                 pltpu.VMEM((NGB, GB), jnp.int32),
                 pltpu.SemaphoreType.DMA]
for _ in range(2):
    _scratch += [pltpu.VMEM((CHUNK, ROWW), jnp.float32),
                 pltpu.VMEM((CHUNK, ROWW), jnp.float32),
                 pltpu.SemaphoreType.DMA]
for _ in range(2):
    _scratch += [pltpu.VMEM((CHUNK, FW), jnp.float32),
                 pltpu.VMEM((CHUNK, FW), jnp.float32),
                 pltpu.SemaphoreType.DMA]
_scratch += [pltpu.VMEM((L,), jnp.float32),
             pltpu.VMEM_SHARED((N_PAD, FW), jnp.float32)]


@functools.partial(
    pl.kernel,
    compiler_params=_cp,
    out_type=(jax.ShapeDtypeStruct((NC, N_PAD, FW), jnp.float32),
              jax.ShapeDtypeStruct((NW, L), jnp.float32)),
    mesh=_mesh,
    scratch_types=_scratch,
)
def _lj_edges(pos_hbm, src_hbm, dst_hbm, zero_hbm, fpart_hbm, epart_hbm,
              *scr):
    idxs = [scr[3 * k:3 * k + 3] for k in range(4)]
    rowss = [scr[12 + 3 * k:12 + 3 * k + 3] for k in range(2)]
    fs = [scr[18 + 3 * k:18 + 3 * k + 3] for k in range(2)]
    eacc = scr[24]
    facc = scr[25]

    cid = lax.axis_index("c")
    sid = lax.axis_index("s")
    wid = sid * NC + cid

    pltpu.sync_copy(zero_hbm.at[pl.ds(sid * RPS, RPS)],
                    facc.at[pl.ds(sid * RPS, RPS)])
    for fp, fn, _ in fs:
        pltpu.sync_copy(zero_hbm.at[pl.ds(0, CHUNK)], fp)
        pltpu.sync_copy(zero_hbm.at[pl.ds(0, CHUNK)], fn)
    eacc[...] = jnp.zeros((L,), jnp.float32)
    plsc.subcore_barrier()

    lane = lax.iota(jnp.int32, L)
    c0 = jnp.zeros((L,), jnp.int32)
    c1 = jnp.full((L,), 1, jnp.int32)
    c2 = jnp.full((L,), 2, jnp.int32)

    row0 = wid * (CPW * NGB)

    def idx_descs(t, si):
        ii, ij, isem = idxs[si]
        r0 = row0 + t * NGB
        return [pltpu.make_async_copy(src_hbm.at[pl.ds(r0, NGB)], ii, isem),
                pltpu.make_async_copy(dst_hbm.at[pl.ds(r0, NGB)], ij, isem)]

    def gather_descs(si, sr):
        ii, ij, _ = idxs[si]
        ri, rj, gsem = rowss[sr]
        ds = []
        for g in range(NGB):
            ds.append(pltpu.make_async_copy(
                pos_hbm.at[ii.at[g]], ri.at[pl.ds(g * GB, GB)], gsem))
            ds.append(pltpu.make_async_copy(
                pos_hbm.at[ij.at[g]], rj.at[pl.ds(g * GB, GB)], gsem))
        return ds

    def issue_scatter(si, sf):
        ii, ij, _ = idxs[si]
        fp, fn, ssem = fs[sf]
        for g in range(NGB):
            pltpu.async_copy(fp.at[pl.ds(g * GB, GB)],
                             facc.at[ij.at[g]], ssem, add=True)
            pltpu.async_copy(fn.at[pl.ds(g * GB, GB)],
                             facc.at[ii.at[g]], ssem, add=True)

    def wait_scatter(sf):
        fp, fn, ssem = fs[sf]
        pltpu.make_async_copy(zero_hbm.at[pl.ds(0, CHUNK)], fp, ssem).wait()
        pltpu.make_async_copy(zero_hbm.at[pl.ds(0, CHUNK)], fn, ssem).wait()

    def compute(sr, sf):
        ri, rj, _ = rowss[sr]
        fp, fn, _ = fs[sf]

        @pl.loop(0, CHUNK, step=L, unroll=2)
        def _grp(e0):
            ridx = lane + e0
            xi = plsc.load_gather(ri, [ridx, c0])
            yi = plsc.load_gather(ri, [ridx, c1])
            zi = plsc.load_gather(ri, [ridx, c2])
            xj = plsc.load_gather(rj, [ridx, c0])
            yj = plsc.load_gather(rj, [ridx, c1])
            zj = plsc.load_gather(rj, [ridx, c2])
            dx = xj - xi
            dy = yj - yi
            dz = zj - zi
            r2 = dx * dx + dy * dy + dz * dz
            s2 = SIG2 / r2
            s6 = s2 * s2 * s2
            s12 = s6 * s6
            mask = r2 < CUT2
            zero = jnp.zeros((L,), jnp.float32)
            e = jnp.where(mask, (4.0 * EPSILON) * (s12 - s6), zero)
            eacc[...] += e
            cf = jnp.where(mask, (FCONST * s2) * (s12 + s12 - s6), zero)
            fx = cf * dx
            fy = cf * dy
            fz = cf * dz
            plsc.store_scatter(fp, [ridx, c0], fx)
            plsc.store_scatter(fp, [ridx, c1], fy)
            plsc.store_scatter(fp, [ridx, c2], fz)
            plsc.store_scatter(fn, [ridx, c0], -fx)
            plsc.store_scatter(fn, [ridx, c1], -fy)
            plsc.store_scatter(fn, [ridx, c2], -fz)

    def phase(t, si, sr, wg1=True, wsc=True, pi2=True):
        if wg1:
            for d in idx_descs(t + 1, (si + 1) % 4):
                d.wait()
            for d in gather_descs((si + 1) % 4, (sr + 1) % 2):
                d.start()
        if wsc:
            wait_scatter(sr)
        if pi2:
            for d in idx_descs(t + 2, (si + 2) % 4):
                d.start()
        for d in gather_descs(si, sr):
            d.wait()
        compute(sr, sr)
        issue_scatter(si, sr)

    for d in idx_descs(0, 0):
        d.start()
        d.wait()
    for d in gather_descs(0, 0):
        d.start()
    for d in idx_descs(1, 1):
        d.start()

    phase(0, 0, 0, wsc=False)
    phase(1, 1, 1, wsc=False)
    phase(2, 2, 0)
    phase(3, 3, 1)

    @pl.loop(4, CPW - 4, step=4)
    def _main(tb):
        phase(tb + 0, 0, 0)
        phase(tb + 1, 1, 1)
        phase(tb + 2, 2, 0)
        phase(tb + 3, 3, 1)

    phase(CPW - 4, 0, 0)
    phase(CPW - 3, 1, 1)
    phase(CPW - 2, 2, 0, pi2=False)
    phase(CPW - 1, 3, 1, wg1=False, pi2=False)
    wait_scatter(0)
    wait_scatter(1)

    plsc.subcore_barrier()
    pltpu.sync_copy(facc.at[pl.ds(sid * RPS, RPS)],
                    fpart_hbm.at[cid, pl.ds(sid * RPS, RPS)])
    pltpu.sync_copy(eacc, epart_hbm.at[wid])


MR = N_PAD * FW // 128


def _merge_body(f_ref, e_ref, o_ref, es_ref):
    o_ref[...] = f_ref[0] + f_ref[1]
    es_ref[...] = jnp.broadcast_to(0.5 * jnp.sum(e_ref[...]), (1, 1))


def kernel(positions, edge_index):
    src = edge_index[0]
    dst = edge_index[1]
    pad = E_PAD - E
    srcp = jnp.concatenate(
        [src, jnp.full((pad,), SENT_I, jnp.int32)]).reshape(-1, GB)
    dstp = jnp.concatenate(
        [dst, jnp.full((pad,), SENT_J, jnp.int32)]).reshape(-1, GB)
    pos16 = jnp.zeros((N_PAD, ROWW), jnp.float32).at[:N, :3].set(positions)
    pos16 = pos16.at[SENT_J, 0].set(1e6)
    z16 = jnp.zeros((N_PAD, FW), jnp.float32)
    fpart, epart = _lj_edges(pos16, srcp, dstp, z16)
    out128, esum = pl.pallas_call(
        _merge_body,
        out_shape=(jax.ShapeDtypeStruct((MR, 128), jnp.float32),
                   jax.ShapeDtypeStruct((1, 1), jnp.float32)),
    )(fpart.reshape(NC, MR, 128), epart)
    forces = out128.reshape(N_PAD, FW)[:N, :3]
    energy = esum[0, 0]
    return energy, forces


# speedup vs baseline: 79.0943x; 1.1595x over previous
import dataclasses
import functools

import jax
import jax.numpy as jnp
from jax import lax
from jax.experimental import pallas as pl
from jax.experimental.pallas import tpu as pltpu
from jax.experimental.pallas import tpu_sc as plsc

N = 100000
E = 6400000
SIGMA = 0.1
EPSILON = 1.0
CUTOFF = 2.5

NC = 2
NS = 16
L = 16
NW = NC * NS
ROWW = 8
FW = 8
GB = 128
NGB = 4
CHUNK = NGB * GB
CPW = 392
E_PAD = NW * CPW * CHUNK
SENT_I = N
SENT_J = N + 1
N_PAD = 100096
RPS = N_PAD // NS

SIG2 = SIGMA * SIGMA
CUT2 = CUTOFF * CUTOFF
FCONST = 24.0 * EPSILON / SIG2

_mesh = plsc.VectorSubcoreMesh(core_axis_name="c", subcore_axis_name="s")

_cp = pltpu.CompilerParams()
if "needs_layout_passes" in pltpu.CompilerParams.__dataclass_fields__:
    _cp = dataclasses.replace(_cp, needs_layout_passes=False)
if "use_tc_tiling_on_sc" in pltpu.CompilerParams.__dataclass_fields__:
    _cp = dataclasses.replace(_cp, use_tc_tiling_on_sc=False)

_scratch = []
for _ in range(4):
    _scratch += [pltpu.VMEM((NGB, GB), jnp.int32),
                 pltpu.VMEM((NGB, GB), jnp.int32),
                 pltpu.SemaphoreType.DMA]
for _ in range(2):
    _scratch += [pltpu.VMEM((CHUNK, ROWW), jnp.float32),
                 pltpu.VMEM((CHUNK, ROWW), jnp.float32),
                 pltpu.SemaphoreType.DMA]
for _ in range(2):
    _scratch += [pltpu.VMEM((CHUNK, FW), jnp.float32),
                 pltpu.VMEM((CHUNK, FW), jnp.float32),
                 pltpu.SemaphoreType.DMA]
_scratch += [pltpu.VMEM((L,), jnp.float32),
             pltpu.VMEM_SHARED((N_PAD, FW), jnp.float32)]


@functools.partial(
    pl.kernel,
    compiler_params=_cp,
    out_type=(jax.ShapeDtypeStruct((NC, N_PAD, FW), jnp.float32),
              jax.ShapeDtypeStruct((NW, L), jnp.float32)),
    mesh=_mesh,
    scratch_types=_scratch,
)
def _lj_edges(pos_hbm, src_hbm, dst_hbm, zero_hbm, fpart_hbm, epart_hbm,
              *scr):
    idxs = [scr[3 * k:3 * k + 3] for k in range(4)]
    rowss = [scr[12 + 3 * k:12 + 3 * k + 3] for k in range(2)]
    fs = [scr[18 + 3 * k:18 + 3 * k + 3] for k in range(2)]
    eacc = scr[24]
    facc = scr[25]

    cid = lax.axis_index("c")
    sid = lax.axis_index("s")
    wid = sid * NC + cid

    pltpu.sync_copy(zero_hbm.at[pl.ds(sid * RPS, RPS)],
                    facc.at[pl.ds(sid * RPS, RPS)])
    for fp, fn, _ in fs:
        pltpu.sync_copy(zero_hbm.at[pl.ds(0, CHUNK)], fp)
        pltpu.sync_copy(zero_hbm.at[pl.ds(0, CHUNK)], fn)
    eacc[...] = jnp.zeros((L,), jnp.float32)
    plsc.subcore_barrier()

    lane = lax.iota(jnp.int32, L)
    c0 = jnp.zeros((L,), jnp.int32)
    c1 = jnp.full((L,), 1, jnp.int32)
    c2 = jnp.full((L,), 2, jnp.int32)

    row0 = wid * (CPW * NGB)

    def idx_descs(t, si):
        ii, ij, isem = idxs[si]
        r0 = row0 + t * NGB
        return [pltpu.make_async_copy(src_hbm.at[pl.ds(r0, NGB)], ii, isem),
                pltpu.make_async_copy(dst_hbm.at[pl.ds(r0, NGB)], ij, isem)]

    def gather_descs(si, sr):
        ii, ij, _ = idxs[si]
        ri, rj, gsem = rowss[sr]
        ds = []
        for g in range(NGB):
            ds.append(pltpu.make_async_copy(
                pos_hbm.at[ii.at[g]], ri.at[pl.ds(g * GB, GB)], gsem))
            ds.append(pltpu.make_async_copy(
                pos_hbm.at[ij.at[g]], rj.at[pl.ds(g * GB, GB)], gsem))
        return ds

    def issue_scatter(si, sf):
        ii, ij, _ = idxs[si]
        fp, fn, ssem = fs[sf]
        for g in range(NGB):
            pltpu.async_copy(fp.at[pl.ds(g * GB, GB)],
                             facc.at[ij.at[g]], ssem, add=True)
            pltpu.async_copy(fn.at[pl.ds(g * GB, GB)],
                             facc.at[ii.at[g]], ssem, add=True)

    def wait_scatter(sf):
        fp, fn, ssem = fs[sf]
        pltpu.make_async_copy(zero_hbm.at[pl.ds(0, CHUNK)], fp, ssem).wait()
        pltpu.make_async_copy(zero_hbm.at[pl.ds(0, CHUNK)], fn, ssem).wait()

    def compute(sr, sf):
        ri, rj, _ = rowss[sr]
        fp, fn, _ = fs[sf]

        @pl.loop(0, CHUNK, step=L, unroll=2)
        def _grp(e0):
            ridx = lane + e0
            xi = plsc.load_gather(ri, [ridx, c0])
            yi = plsc.load_gather(ri, [ridx, c1])
            zi = plsc.load_gather(ri, [ridx, c2])
            xj = plsc.load_gather(rj, [ridx, c0])
            yj = plsc.load_gather(rj, [ridx, c1])
            zj = plsc.load_gather(rj, [ridx, c2])
            dx = xj - xi
            dy = yj - yi
            dz = zj - zi
            r2 = dx * dx + dy * dy + dz * dz
            s2 = SIG2 / r2
            s6 = s2 * s2 * s2
            s12 = s6 * s6
            mask = r2 < CUT2
            zero = jnp.zeros((L,), jnp.float32)
            e = jnp.where(mask, (4.0 * EPSILON) * (s12 - s6), zero)
            eacc[...] += e
            cf = jnp.where(mask, (FCONST * s2) * (s12 + s12 - s6), zero)
            fx = cf * dx
            fy = cf * dy
            fz = cf * dz
            plsc.store_scatter(fp, [ridx, c0], fx)
            plsc.store_scatter(fp, [ridx, c1], fy)
            plsc.store_scatter(fp, [ridx, c2], fz)
            plsc.store_scatter(fn, [ridx, c0], -fx)
            plsc.store_scatter(fn, [ridx, c1], -fy)
            plsc.store_scatter(fn, [ridx, c2], -fz)

    def phase(t, si, sr, wg1=True, wsc=True, pi2=True):
        if wg1:
            for d in idx_descs(t + 1, (si + 1) % 4):
                d.wait()
            for d in gather_descs((si + 1) % 4, (sr + 1) % 2):
                d.start()
        if wsc:
            wait_scatter(sr)
        if pi2:
            for d in idx_descs(t + 2, (si + 2) % 4):
                d.start()
        for d in gather_descs(si, sr):
            d.wait()
        compute(sr, sr)
        issue_scatter(si, sr)

    for d in idx_descs(0, 0):
        d.start()
        d.wait()
    for d in gather_descs(0, 0):
        d.start()
    for d in idx_descs(1, 1):
        d.start()

    phase(0, 0, 0, wsc=False)
    phase(1, 1, 1, wsc=False)
    phase(2, 2, 0)
    phase(3, 3, 1)

    @pl.loop(4, CPW - 4, step=4)
    def _main(tb):
        phase(tb + 0, 0, 0)
        phase(tb + 1, 1, 1)
        phase(tb + 2, 2, 0)
        phase(tb + 3, 3, 1)

    phase(CPW - 4, 0, 0)
    phase(CPW - 3, 1, 1)
    phase(CPW - 2, 2, 0, pi2=False)
    phase(CPW - 1, 3, 1, wg1=False, pi2=False)
    wait_scatter(0)
    wait_scatter(1)

    plsc.subcore_barrier()
    pltpu.sync_copy(facc.at[pl.ds(sid * RPS, RPS)],
                    fpart_hbm.at[cid, pl.ds(sid * RPS, RPS)])
    pltpu.sync_copy(eacc, epart_hbm.at[wid])


MR = N_PAD * FW // 128


def _merge_body(f_ref, e_ref, o_ref, es_ref):
    o_ref[...] = f_ref[0] + f_ref[1]
    es_ref[...] = jnp.broadcast_to(0.5 * jnp.sum(e_ref[...]), (1, 1))


def kernel(positions, edge_index):
    src = edge_index[0]
    dst = edge_index[1]
    pad = E_PAD - E
    srcp = jnp.concatenate(
        [src, jnp.full((pad,), SENT_I, jnp.int32)]).reshape(-1, GB)
    dstp = jnp.concatenate(
        [dst, jnp.full((pad,), SENT_J, jnp.int32)]).reshape(-1, GB)
    pos16 = jnp.zeros((N_PAD, ROWW), jnp.float32).at[:N, :3].set(positions)
    pos16 = pos16.at[SENT_J, 0].set(1e6)
    z16 = jnp.zeros((N_PAD, FW), jnp.float32)
    fpart, epart = _lj_edges(pos16, srcp, dstp, z16)
    out128, esum = pl.pallas_call(
        _merge_body,
        out_shape=(jax.ShapeDtypeStruct((MR, 128), jnp.float32),
                   jax.ShapeDtypeStruct((1, 1), jnp.float32)),
    )(fpart.reshape(NC, MR, 128), epart)
    forces = out128.reshape(N_PAD, FW)[:N, :3]
    energy = esum[0, 0]
    return energy, forces


# speedup vs baseline: 95.9611x; 1.2132x over previous
import dataclasses
import functools

import jax
import jax.numpy as jnp
from jax import lax
from jax.experimental import pallas as pl
from jax.experimental.pallas import tpu as pltpu
from jax.experimental.pallas import tpu_sc as plsc

N = 100000
E = 6400000
SIGMA = 0.1
EPSILON = 1.0
CUTOFF = 2.5

NC = 2
NS = 16
L = 16
NW = NC * NS
ROWW = 8
FW = 8
GB = 128
NGB = 4
CHUNK = NGB * GB
CPW = 392
E_PAD = NW * CPW * CHUNK
SENT_I = N
SENT_J = N + 1
N_PAD = 100096
HPAD = N_PAD // 2
RPS = N_PAD // NS
HRPS = HPAD // NS

SIG2 = SIGMA * SIGMA
CUT2 = CUTOFF * CUTOFF
FCONST = 24.0 * EPSILON / SIG2

_mesh = plsc.VectorSubcoreMesh(core_axis_name="c", subcore_axis_name="s")

_cp = pltpu.CompilerParams()
if "needs_layout_passes" in pltpu.CompilerParams.__dataclass_fields__:
    _cp = dataclasses.replace(_cp, needs_layout_passes=False)
if "use_tc_tiling_on_sc" in pltpu.CompilerParams.__dataclass_fields__:
    _cp = dataclasses.replace(_cp, use_tc_tiling_on_sc=False)

_scratch = []
for _ in range(4):
    _scratch += [pltpu.VMEM((NGB, GB), jnp.int32),
                 pltpu.VMEM((NGB, GB), jnp.int32),
                 pltpu.VMEM((NGB, GB), jnp.int32),
                 pltpu.VMEM((NGB, GB), jnp.int32),
                 pltpu.SemaphoreType.DMA]
for _ in range(2):
    _scratch += [pltpu.VMEM((CHUNK, ROWW), jnp.float32),
                 pltpu.VMEM((CHUNK, ROWW), jnp.float32),
                 pltpu.SemaphoreType.DMA]
for _ in range(2):
    _scratch += [pltpu.VMEM((CHUNK, FW), jnp.float32),
                 pltpu.VMEM((CHUNK, FW), jnp.float32),
                 pltpu.SemaphoreType.DMA]
_scratch += [pltpu.VMEM((L,), jnp.float32),
             pltpu.VMEM_SHARED((HPAD, FW), jnp.float32),
             pltpu.VMEM_SHARED((N_PAD, ROWW), jnp.float32)]


@functools.partial(
    pl.kernel,
    compiler_params=_cp,
    out_type=(jax.ShapeDtypeStruct((NC, HPAD, FW), jnp.float32),
              jax.ShapeDtypeStruct((NW, L), jnp.float32)),
    mesh=_mesh,
    scratch_types=_scratch,
)
def _lj_edges(pos_hbm, src_hbm, dst_hbm, fpart_hbm, epart_hbm, *scr):
    idxs = [scr[5 * k:5 * k + 5] for k in range(4)]
    rowss = [scr[20 + 3 * k:20 + 3 * k + 3] for k in range(2)]
    fs = [scr[26 + 3 * k:26 + 3 * k + 3] for k in range(2)]
    eacc = scr[32]
    facc = scr[33]
    pos_spm = scr[34]

    cid = lax.axis_index("c")
    sid = lax.axis_index("s")
    wid = sid * NC + cid

    lane = lax.iota(jnp.int32, L)

    zvec = jnp.zeros((L,), jnp.float32)
    fp0 = fs[0][0]
    for buf in [fs[0][0], fs[0][1], fs[1][0], fs[1][1]]:
        @pl.loop(0, CHUNK * FW, step=L)
        def _zero(k, buf=buf):
            flat = lane + k
            plsc.store_scatter(
                buf, [lax.shift_right_logical(flat, 3),
                      lax.bitwise_and(flat, 7)], zvec)
    sbase = sid * HRPS
    for q in range(HRPS // CHUNK):
        pltpu.sync_copy(fp0, facc.at[pl.ds(sbase + q * CHUNK, CHUNK)])
    rem = HRPS % CHUNK
    if rem:
        pltpu.sync_copy(fp0.at[pl.ds(0, rem)],
                        facc.at[pl.ds(sbase + (HRPS // CHUNK) * CHUNK, rem)])
    pltpu.sync_copy(pos_hbm.at[pl.ds(sid * RPS, RPS)],
                    pos_spm.at[pl.ds(sid * RPS, RPS)])
    eacc[...] = jnp.zeros((L,), jnp.float32)
    plsc.subcore_barrier()

    c0 = jnp.zeros((L,), jnp.int32)
    c1 = jnp.full((L,), 1, jnp.int32)
    c2 = jnp.full((L,), 2, jnp.int32)
    ones = jnp.full((L,), 1, jnp.int32)
    fours = jnp.full((L,), 4, jnp.int32)

    row0 = wid * (CPW * NGB)

    def idx_descs(t, si):
        ii, ij, _, _, isem = idxs[si]
        r0 = row0 + t * NGB
        return [pltpu.make_async_copy(src_hbm.at[pl.ds(r0, NGB)], ii, isem),
                pltpu.make_async_copy(dst_hbm.at[pl.ds(r0, NGB)], ij, isem)]

    def gather_descs(si, sr):
        ii, ij, _, _, _ = idxs[si]
        ri, rj, gsem = rowss[sr]
        ds = []
        for g in range(NGB):
            ds.append(pltpu.make_async_copy(
                pos_spm.at[ii.at[g]], ri.at[pl.ds(g * GB, GB)], gsem))
            ds.append(pltpu.make_async_copy(
                pos_spm.at[ij.at[g]], rj.at[pl.ds(g * GB, GB)], gsem))
        return ds

    def issue_scatter(si, sf):
        _, _, si_r, sj_r, _ = idxs[si]
        fp, fn, ssem = fs[sf]
        for g in range(NGB):
            pltpu.async_copy(fp.at[pl.ds(g * GB, GB)],
                             facc.at[sj_r.at[g]], ssem, add=True)
            pltpu.async_copy(fn.at[pl.ds(g * GB, GB)],
                             facc.at[si_r.at[g]], ssem, add=True)

    def wait_scatter(sf):
        fp, fn, ssem = fs[sf]
        pltpu.make_async_copy(pos_hbm.at[pl.ds(0, CHUNK)], fp, ssem).wait()
        pltpu.make_async_copy(pos_hbm.at[pl.ds(0, CHUNK)], fn, ssem).wait()

    def compute(si, sr, sf):
        ii, ij, si_r, sj_r, _ = idxs[si]
        ri, rj, _ = rowss[sr]
        fp, fn, _ = fs[sf]

        @pl.loop(0, CHUNK, step=L, unroll=2)
        def _grp(e0):
            ridx = lane + e0
            gv = lane * 0 + lax.shift_right_logical(e0, 7)
            cv = lane + lax.bitwise_and(e0, 127)
            ni = plsc.load_gather(ii, [gv, cv])
            nj = plsc.load_gather(ij, [gv, cv])
            plsc.store_scatter(si_r, [gv, cv],
                               lax.shift_right_logical(ni, 1))
            plsc.store_scatter(sj_r, [gv, cv],
                               lax.shift_right_logical(nj, 1))
            cpi = lax.shift_left(lax.bitwise_and(ni, ones), 2)
            cpj = lax.shift_left(lax.bitwise_and(nj, ones), 2)
            cqi = fours - cpi
            cqj = fours - cpj
            xi = plsc.load_gather(ri, [ridx, c0])
            yi = plsc.load_gather(ri, [ridx, c1])
            zi = plsc.load_gather(ri, [ridx, c2])
            xj = plsc.load_gather(rj, [ridx, c0])
            yj = plsc.load_gather(rj, [ridx, c1])
            zj = plsc.load_gather(rj, [ridx, c2])
            dx = xj - xi
            dy = yj - yi
            dz = zj - zi
            r2 = dx * dx + dy * dy + dz * dz
            s2 = SIG2 / r2
            s6 = s2 * s2 * s2
            s12 = s6 * s6
            mask = r2 < CUT2
            zero = jnp.zeros((L,), jnp.float32)
            e = jnp.where(mask, (4.0 * EPSILON) * (s12 - s6), zero)
            eacc[...] += e
            cf = jnp.where(mask, (FCONST * s2) * (s12 + s12 - s6), zero)
            fx = cf * dx
            fy = cf * dy
            fz = cf * dz
            plsc.store_scatter(fp, [ridx, cpj], fx)
            plsc.store_scatter(fp, [ridx, cpj + c1], fy)
            plsc.store_scatter(fp, [ridx, cpj + c2], fz)
            plsc.store_scatter(fp, [ridx, cqj], zero)
            plsc.store_scatter(fp, [ridx, cqj + c1], zero)
            plsc.store_scatter(fp, [ridx, cqj + c2], zero)
            plsc.store_scatter(fn, [ridx, cpi], -fx)
            plsc.store_scatter(fn, [ridx, cpi + c1], -fy)
            plsc.store_scatter(fn, [ridx, cpi + c2], -fz)
            plsc.store_scatter(fn, [ridx, cqi], zero)
            plsc.store_scatter(fn, [ridx, cqi + c1], zero)
            plsc.store_scatter(fn, [ridx, cqi + c2], zero)

    def phase(t, si, sr, wg1=True, wsc=True, pi2=True):
        if wg1:
            for d in idx_descs(t + 1, (si + 1) % 4):
                d.wait()
            for d in gather_descs((si + 1) % 4, (sr + 1) % 2):
                d.start()
        if wsc:
            wait_scatter(sr)
        if pi2:
            for d in idx_descs(t + 2, (si + 2) % 4):
                d.start()
        for d in gather_descs(si, sr):
            d.wait()
        compute(si, sr, sr)
        issue_scatter(si, sr)

    for d in idx_descs(0, 0):
        d.start()
        d.wait()
    for d in gather_descs(0, 0):
        d.start()
    for d in idx_descs(1, 1):
        d.start()

    phase(0, 0, 0, wsc=False)
    phase(1, 1, 1, wsc=False)
    phase(2, 2, 0)
    phase(3, 3, 1)

    @pl.loop(4, CPW - 4, step=4)
    def _main(tb):
        phase(tb + 0, 0, 0)
        phase(tb + 1, 1, 1)
        phase(tb + 2, 2, 0)
        phase(tb + 3, 3, 1)

    phase(CPW - 4, 0, 0)
    phase(CPW - 3, 1, 1)
    phase(CPW - 2, 2, 0, pi2=False)
    phase(CPW - 1, 3, 1, wg1=False, pi2=False)
    wait_scatter(0)
    wait_scatter(1)

    plsc.subcore_barrier()
    pltpu.sync_copy(facc.at[pl.ds(sid * HRPS, HRPS)],
                    fpart_hbm.at[cid, pl.ds(sid * HRPS, HRPS)])
    pltpu.sync_copy(eacc, epart_hbm.at[wid])


MR = HPAD * FW // 128


def _merge_body(f_ref, e_ref, o_ref, es_ref):
    o_ref[...] = f_ref[0] + f_ref[1]
    es_ref[...] = jnp.broadcast_to(0.5 * jnp.sum(e_ref[...]), (1, 1))


def kernel(positions, edge_index):
    src = edge_index[0]
    dst = edge_index[1]
    pad = E_PAD - E
    srcp = jnp.concatenate(
        [src, jnp.full((pad,), SENT_I, jnp.int32)]).reshape(-1, GB)
    dstp = jnp.concatenate(
        [dst, jnp.full((pad,), SENT_J, jnp.int32)]).reshape(-1, GB)
    pos16 = jnp.zeros((N_PAD, ROWW), jnp.float32).at[:N, :3].set(positions)
    pos16 = pos16.at[SENT_J, 0].set(1e6)
    fpart, epart = _lj_edges(pos16, srcp, dstp)
    out128, esum = pl.pallas_call(
        _merge_body,
        out_shape=(jax.ShapeDtypeStruct((MR, 128), jnp.float32),
                   jax.ShapeDtypeStruct((1, 1), jnp.float32)),
    )(fpart.reshape(NC, MR, 128), epart)
    forces = out128.reshape(N_PAD, 4)[:N, :3]
    energy = esum[0, 0]
    return energy, forces


# speedup vs baseline: 102.3109x; 1.0662x over previous
import dataclasses
import functools

import jax
import jax.numpy as jnp
from jax import lax
from jax.experimental import pallas as pl
from jax.experimental.pallas import tpu as pltpu
from jax.experimental.pallas import tpu_sc as plsc

N = 100000
E = 6400000
SIGMA = 0.1
EPSILON = 1.0
CUTOFF = 2.5

NC = 2
NS = 16
L = 16
NW = NC * NS
ROWW = 8
FW = 8
GB = 128
NGB = 4
CHUNK = NGB * GB
CPW = 392
E_PAD = NW * CPW * CHUNK
ER = E // GB
PR = (E_PAD - E) // GB
SENT_I = N
SENT_J = N + 1
N_PAD = 100096
HPAD = N_PAD // 2
RPS = N_PAD // NS
HRPS = HPAD // NS

SIG2 = SIGMA * SIGMA
CUT2 = CUTOFF * CUTOFF
FCONST = 24.0 * EPSILON / SIG2

_mesh = plsc.VectorSubcoreMesh(core_axis_name="c", subcore_axis_name="s")

_cp = pltpu.CompilerParams()
if "needs_layout_passes" in pltpu.CompilerParams.__dataclass_fields__:
    _cp = dataclasses.replace(_cp, needs_layout_passes=False)
if "use_tc_tiling_on_sc" in pltpu.CompilerParams.__dataclass_fields__:
    _cp = dataclasses.replace(_cp, use_tc_tiling_on_sc=False)

_scratch = []
for _ in range(4):
    _scratch += [pltpu.VMEM((NGB, GB), jnp.int32),
                 pltpu.VMEM((NGB, GB), jnp.int32),
                 pltpu.VMEM((NGB, GB), jnp.int32),
                 pltpu.VMEM((NGB, GB), jnp.int32),
                 pltpu.SemaphoreType.DMA]
for _ in range(2):
    _scratch += [pltpu.VMEM((CHUNK, ROWW), jnp.float32),
                 pltpu.VMEM((CHUNK, ROWW), jnp.float32),
                 pltpu.SemaphoreType.DMA]
for _ in range(2):
    _scratch += [pltpu.VMEM((CHUNK, FW), jnp.float32),
                 pltpu.VMEM((CHUNK, FW), jnp.float32),
                 pltpu.SemaphoreType.DMA]
_scratch += [pltpu.VMEM((L,), jnp.float32),
             pltpu.VMEM_SHARED((HPAD, FW), jnp.float32),
             pltpu.VMEM_SHARED((N_PAD, ROWW), jnp.float32)]


@functools.partial(
    pl.kernel,
    compiler_params=_cp,
    out_type=(jax.ShapeDtypeStruct((NC, HPAD, FW), jnp.float32),
              jax.ShapeDtypeStruct((NW, L), jnp.float32)),
    mesh=_mesh,
    scratch_types=_scratch,
)
def _lj_edges(pos_hbm, ei_hbm, padi_hbm, padj_hbm, fpart_hbm, epart_hbm,
              *scr):
    idxs = [scr[5 * k:5 * k + 5] for k in range(4)]
    rowss = [scr[20 + 3 * k:20 + 3 * k + 3] for k in range(2)]
    fs = [scr[26 + 3 * k:26 + 3 * k + 3] for k in range(2)]
    eacc = scr[32]
    facc = scr[33]
    pos_spm = scr[34]

    cid = lax.axis_index("c")
    sid = lax.axis_index("s")
    wid = sid * NC + cid

    lane = lax.iota(jnp.int32, L)

    zvec = jnp.zeros((L,), jnp.float32)
    fp0 = fs[0][0]
    for buf in [fs[0][0], fs[0][1], fs[1][0], fs[1][1]]:
        @pl.loop(0, CHUNK * FW, step=L)
        def _zero(k, buf=buf):
            flat = lane + k
            plsc.store_scatter(
                buf, [lax.shift_right_logical(flat, 3),
                      lax.bitwise_and(flat, 7)], zvec)
    sbase = sid * HRPS
    for q in range(HRPS // CHUNK):
        pltpu.sync_copy(fp0, facc.at[pl.ds(sbase + q * CHUNK, CHUNK)])
    rem = HRPS % CHUNK
    if rem:
        pltpu.sync_copy(fp0.at[pl.ds(0, rem)],
                        facc.at[pl.ds(sbase + (HRPS // CHUNK) * CHUNK, rem)])
    pltpu.sync_copy(pos_hbm.at[pl.ds(sid * RPS, RPS)],
                    pos_spm.at[pl.ds(sid * RPS, RPS)])
    eacc[...] = jnp.zeros((L,), jnp.float32)
    plsc.subcore_barrier()

    c0 = jnp.zeros((L,), jnp.int32)
    c1 = jnp.full((L,), 1, jnp.int32)
    c2 = jnp.full((L,), 2, jnp.int32)
    ones = jnp.full((L,), 1, jnp.int32)
    fours = jnp.full((L,), 4, jnp.int32)

    row0 = wid * (CPW * NGB)

    def issue_idx(t, si):
        ii, ij, _, _, isem = idxs[si]
        r0 = row0 + t * NGB

        @pl.when(r0 < ER)
        def _():
            pltpu.make_async_copy(
                ei_hbm.at[0, pl.ds(r0, NGB)], ii, isem).start()
            pltpu.make_async_copy(
                ei_hbm.at[1, pl.ds(r0, NGB)], ij, isem).start()

        @pl.when(r0 >= ER)
        def _():
            rp = r0 - ER
            pltpu.make_async_copy(
                padi_hbm.at[pl.ds(rp, NGB)], ii, isem).start()
            pltpu.make_async_copy(
                padj_hbm.at[pl.ds(rp, NGB)], ij, isem).start()

    def wait_idx(si):
        ii, ij, _, _, isem = idxs[si]
        pltpu.make_async_copy(padi_hbm.at[pl.ds(0, NGB)], ii, isem).wait()
        pltpu.make_async_copy(padj_hbm.at[pl.ds(0, NGB)], ij, isem).wait()

    def gather_descs(si, sr):
        ii, ij, _, _, _ = idxs[si]
        ri, rj, gsem = rowss[sr]
        ds = []
        for g in range(NGB):
            ds.append(pltpu.make_async_copy(
                pos_spm.at[ii.at[g]], ri.at[pl.ds(g * GB, GB)], gsem))
            ds.append(pltpu.make_async_copy(
                pos_spm.at[ij.at[g]], rj.at[pl.ds(g * GB, GB)], gsem))
        return ds

    def issue_scatter(si, sf):
        _, _, si_r, sj_r, _ = idxs[si]
        fp, fn, ssem = fs[sf]
        for g in range(NGB):
            pltpu.async_copy(fp.at[pl.ds(g * GB, GB)],
                             facc.at[sj_r.at[g]], ssem, add=True)
            pltpu.async_copy(fn.at[pl.ds(g * GB, GB)],
                             facc.at[si_r.at[g]], ssem, add=True)

    def wait_scatter(sf):
        fp, fn, ssem = fs[sf]
        pltpu.make_async_copy(pos_hbm.at[pl.ds(0, CHUNK)], fp, ssem).wait()
        pltpu.make_async_copy(pos_hbm.at[pl.ds(0, CHUNK)], fn, ssem).wait()

    def compute(si, sr, sf):
        ii, ij, si_r, sj_r, _ = idxs[si]
        ri, rj, _ = rowss[sr]
        fp, fn, _ = fs[sf]

        @pl.loop(0, CHUNK, step=L, unroll=2)
        def _grp(e0):
            ridx = lane + e0
            gv = lane * 0 + lax.shift_right_logical(e0, 7)
            cv = lane + lax.bitwise_and(e0, 127)
            ni = plsc.load_gather(ii, [gv, cv])
            nj = plsc.load_gather(ij, [gv, cv])
            plsc.store_scatter(si_r, [gv, cv],
                               lax.shift_right_logical(ni, 1))
            plsc.store_scatter(sj_r, [gv, cv],
                               lax.shift_right_logical(nj, 1))
            cpi = lax.shift_left(lax.bitwise_and(ni, ones), 2)
            cpj = lax.shift_left(lax.bitwise_and(nj, ones), 2)
            cqi = fours - cpi
            cqj = fours - cpj
            xi = plsc.load_gather(ri, [ridx, c0])
            yi = plsc.load_gather(ri, [ridx, c1])
            zi = plsc.load_gather(ri, [ridx, c2])
            xj = plsc.load_gather(rj, [ridx, c0])
            yj = plsc.load_gather(rj, [ridx, c1])
            zj = plsc.load_gather(rj, [ridx, c2])
            dx = xj - xi
            dy = yj - yi
            dz = zj - zi
            r2 = dx * dx + dy * dy + dz * dz
            s2 = SIG2 / r2
            s6 = s2 * s2 * s2
            s12 = s6 * s6
            mask = r2 < CUT2
            zero = jnp.zeros((L,), jnp.float32)
            e = jnp.where(mask, (4.0 * EPSILON) * (s12 - s6), zero)
            eacc[...] += e
            cf = jnp.where(mask, (FCONST * s2) * (s12 + s12 - s6), zero)
            fx = cf * dx
            fy = cf * dy
            fz = cf * dz
            plsc.store_scatter(fp, [ridx, cpj], fx)
            plsc.store_scatter(fp, [ridx, cpj + c1], fy)
            plsc.store_scatter(fp, [ridx, cpj + c2], fz)
            plsc.store_scatter(fp, [ridx, cqj], zero)
            plsc.store_scatter(fp, [ridx, cqj + c1], zero)
            plsc.store_scatter(fp, [ridx, cqj + c2], zero)
            plsc.store_scatter(fn, [ridx, cpi], -fx)
            plsc.store_scatter(fn, [ridx, cpi + c1], -fy)
            plsc.store_scatter(fn, [ridx, cpi + c2], -fz)
            plsc.store_scatter(fn, [ridx, cqi], zero)
            plsc.store_scatter(fn, [ridx, cqi + c1], zero)
            plsc.store_scatter(fn, [ridx, cqi + c2], zero)

    def phase(t, si, sr, wg1=True, wsc=True, pi2=True):
        if wg1:
            wait_idx((si + 1) % 4)
            for d in gather_descs((si + 1) % 4, (sr + 1) % 2):
                d.start()
        if wsc:
            wait_scatter(sr)
        if pi2:
            issue_idx(t + 2, (si + 2) % 4)
        for d in gather_descs(si, sr):
            d.wait()
        compute(si, sr, sr)
        issue_scatter(si, sr)

    issue_idx(0, 0)
    wait_idx(0)
    for d in gather_descs(0, 0):
        d.start()
    issue_idx(1, 1)

    phase(0, 0, 0, wsc=False)
    phase(1, 1, 1, wsc=False)
    phase(2, 2, 0)
    phase(3, 3, 1)

    @pl.loop(4, CPW - 4, step=4)
    def _main(tb):
        phase(tb + 0, 0, 0)
        phase(tb + 1, 1, 1)
        phase(tb + 2, 2, 0)
        phase(tb + 3, 3, 1)

    phase(CPW - 4, 0, 0)
    phase(CPW - 3, 1, 1)
    phase(CPW - 2, 2, 0, pi2=False)
    phase(CPW - 1, 3, 1, wg1=False, pi2=False)
    wait_scatter(0)
    wait_scatter(1)

    plsc.subcore_barrier()
    pltpu.sync_copy(facc.at[pl.ds(sid * HRPS, HRPS)],
                    fpart_hbm.at[cid, pl.ds(sid * HRPS, HRPS)])
    pltpu.sync_copy(eacc, epart_hbm.at[wid])


MR = HPAD * FW // 128


def _merge_body(f_ref, e_ref, o_ref, es_ref):
    o_ref[...] = f_ref[0] + f_ref[1]
    es_ref[...] = jnp.broadcast_to(0.5 * jnp.sum(e_ref[...]), (1, 1))


def kernel(positions, edge_index):
    ei3 = edge_index.reshape(2, ER, GB)
    padi = jnp.full((PR, GB), SENT_I, jnp.int32)
    padj = jnp.full((PR, GB), SENT_J, jnp.int32)
    pos16 = jnp.zeros((N_PAD, ROWW), jnp.float32).at[:N, :3].set(positions)
    pos16 = pos16.at[SENT_J, 0].set(1e6)
    fpart, epart = _lj_edges(pos16, ei3, padi, padj)
    out128, esum = pl.pallas_call(
        _merge_body,
        out_shape=(jax.ShapeDtypeStruct((MR, 128), jnp.float32),
                   jax.ShapeDtypeStruct((1, 1), jnp.float32)),
    )(fpart.reshape(NC, MR, 128), epart)
    forces = out128.reshape(N_PAD, 4)[:N, :3]
    energy = esum[0, 0]
    return energy, forces
